# Initial kernel scaffold; baseline (speedup 1.0000x reference)
import jax
import jax.numpy as jnp
from jax.experimental import pallas as pl


def kernel(lin_w, lin_b, bn0_g, bn0_b, w0, w0_sub, b0, bn1_g, bn1_b, w1, w1_sub, b1, bn2_g, bn2_b, w2, w2_sub, b2, z):
    raise NotImplementedError("write your pallas kernel here")


# speedup vs baseline: 6.9412x; 6.9412x over previous
import functools

import jax
import jax.numpy as jnp
from jax.experimental import pallas as pl
from jax.experimental.pallas import tpu as pltpu

_BN_EPS = 1e-5
_TAPS = ((0, 0), (0, 1), (1, 0), (1, 1))


def _linear_kernel(z_ref, w_ref, b_ref, g_ref, beta_ref, o_ref, *, batch):
    acc = jnp.dot(z_ref[...], w_ref[...], preferred_element_type=jnp.float32)
    acc = acc + b_ref[...]
    inv_n = 1.0 / batch
    mean = jnp.sum(acc, axis=0, keepdims=True) * inv_n
    var = jnp.sum((acc - mean) ** 2, axis=0, keepdims=True) * inv_n
    scale = g_ref[...] * jax.lax.rsqrt(var + _BN_EPS)
    shift = beta_ref[...] - mean * scale
    o_ref[...] = jnp.maximum(acc * scale + shift, 0.0)


def _linear_bn_relu(z, w, b, gamma, beta):
    B, K = z.shape
    N = w.shape[1]
    nb = 2 if N % 2 == 0 else 1
    Nb = N // nb
    return pl.pallas_call(
        functools.partial(_linear_kernel, batch=float(B)),
        out_shape=jax.ShapeDtypeStruct((B, N), jnp.float32),
        grid=(nb,),
        in_specs=[
            pl.BlockSpec((B, K), lambda i: (0, 0)),
            pl.BlockSpec((K, Nb), lambda i: (0, i)),
            pl.BlockSpec((1, Nb), lambda i: (0, i)),
            pl.BlockSpec((1, Nb), lambda i: (0, i)),
            pl.BlockSpec((1, Nb), lambda i: (0, i)),
        ],
        out_specs=pl.BlockSpec((B, Nb), lambda i: (0, i)),
        compiler_params=pltpu.CompilerParams(dimension_semantics=("parallel",)),
    )(z, w, b.reshape(1, N), gamma.reshape(1, N), beta.reshape(1, N))


def _shift_mask(y, dyx, H, W, lane):
    dy, dx = dyx
    s = dy * W + dx
    out = jnp.roll(y, -s, axis=1) if s != 0 else y
    valid = None
    if dy or dx:
        px = lane % W
        py = (lane // W) % H
        conds = []
        if dy:
            conds.append((py + dy >= 0) if dy < 0 else (py + dy < H))
        if dx:
            conds.append((px + dx >= 0) if dx < 0 else (px + dx < W))
        valid = conds[0]
        for c in conds[1:]:
            valid = valid & c
        out = jnp.where(valid, out, 0.0)
    return out


def _convt_kernel(x_ref, w_ref, b_ref, g_ref, beta_ref, o_ref, *,
                  H, W, cin, use_bn, act, n_total):
    x = x_ref[...].astype(jnp.bfloat16)
    M = x.shape[1]
    lane = jax.lax.broadcasted_iota(jnp.int32, (1, M), 1)
    accs = []
    for r in range(4):
        ry, rx = r // 2, r % 2
        wr = w_ref[r]
        acc = None
        for t, (dy, dx) in enumerate(_TAPS):
            wt = wr[:, t * cin:(t + 1) * cin].astype(jnp.bfloat16)
            y = jnp.dot(wt, x, preferred_element_type=jnp.float32)
            y = _shift_mask(y, (ry + dy - 1, rx + dx - 1), H, W, lane)
            acc = y if acc is None else acc + y
        accs.append(acc + b_ref[...])

    if use_bn:
        inv_n = 1.0 / n_total
        total = accs[0].sum(axis=1, keepdims=True)
        for r in range(1, 4):
            total = total + accs[r].sum(axis=1, keepdims=True)
        mean = total * inv_n
        sq = ((accs[0] - mean) ** 2).sum(axis=1, keepdims=True)
        for r in range(1, 4):
            sq = sq + ((accs[r] - mean) ** 2).sum(axis=1, keepdims=True)
        var = sq * inv_n
        scale = g_ref[...] * jax.lax.rsqrt(var + _BN_EPS)
        shift = beta_ref[...] - mean * scale
        accs = [a * scale + shift for a in accs]

    if act == "relu":
        accs = [jnp.maximum(a, 0.0) for a in accs]

    for r in range(4):
        o_ref[r, :, :] = accs[r]


def _convt_bn_relu(x, w_sub, b, gamma, beta, *, H, use_bn, act):
    Cin, M = x.shape
    Cout = w_sub.shape[1]
    nb = 2 if Cout % 2 == 0 else 1
    Cb = Cout // nb
    return pl.pallas_call(
        functools.partial(_convt_kernel, H=H, W=H, cin=Cin, use_bn=use_bn,
                          act=act, n_total=float(4 * M)),
        out_shape=jax.ShapeDtypeStruct((4, Cout, M), jnp.float32),
        grid=(nb,),
        in_specs=[
            pl.BlockSpec((Cin, M), lambda i: (0, 0)),
            pl.BlockSpec((4, Cb, 4 * Cin), lambda i: (0, i, 0)),
            pl.BlockSpec((Cb, 1), lambda i: (i, 0)),
            pl.BlockSpec((Cb, 1), lambda i: (i, 0)),
            pl.BlockSpec((Cb, 1), lambda i: (i, 0)),
        ],
        out_specs=pl.BlockSpec((4, Cb, M), lambda i: (0, i, 0)),
        compiler_params=pltpu.CompilerParams(dimension_semantics=("parallel",)),
    )(x, w_sub, b.reshape(Cout, 1), gamma.reshape(Cout, 1), beta.reshape(Cout, 1))


def _conv_last_kernel(x_ref, w_ref, o_ref, *, H, W, cin):
    x = x_ref[...].astype(jnp.bfloat16)
    M = x.shape[1]
    lane = jax.lax.broadcasted_iota(jnp.int32, (1, M), 1)
    wall = w_ref[:16, :].astype(jnp.bfloat16)
    ys = jnp.dot(wall, x, preferred_element_type=jnp.float32)
    bias = w_ref[16, 0]
    for r in range(4):
        ry, rx = r // 2, r % 2
        acc = None
        for t, (dy, dx) in enumerate(_TAPS):
            y = _shift_mask(ys[4 * r + t:4 * r + t + 1, :],
                            (ry + dy - 1, rx + dx - 1), H, W, lane)
            acc = y if acc is None else acc + y
        o_ref[r:r + 1, :] = jnp.tanh(acc + bias)


def _conv_last(x, w_sub, b, *, H):
    Cin, M = x.shape
    w16 = w_sub.reshape(16, Cin)
    wpack = jnp.concatenate([w16, jnp.full((1, Cin), 0.0) + b], axis=0)
    nb = 2 if M % 2 == 0 else 1
    Mb = M // nb
    return pl.pallas_call(
        functools.partial(_conv_last_kernel, H=H, W=H, cin=Cin),
        out_shape=jax.ShapeDtypeStruct((4, M), jnp.float32),
        grid=(nb,),
        in_specs=[
            pl.BlockSpec((Cin, Mb), lambda i: (0, i)),
            pl.BlockSpec((17, Cin), lambda i: (0, 0)),
        ],
        out_specs=pl.BlockSpec((4, Mb), lambda i: (0, i)),
        compiler_params=pltpu.CompilerParams(dimension_semantics=("parallel",)),
    )(x, wpack)


def _interleave(planes, B, H):
    C = planes.shape[1]
    o = planes.reshape(2, 2, C, B, H, H)
    o = jnp.transpose(o, (2, 3, 4, 0, 5, 1))
    return o.reshape(C, B * 4 * H * H)


def kernel(lin_w, lin_b, bn0_g, bn0_b, w0, w0_sub, b0, bn1_g, bn1_b,
           w1, w1_sub, b1, bn2_g, bn2_b, w2, w2_sub, b2, z):
    B = z.shape[0]
    lin_out = lin_w.shape[1]
    C1 = lin_out // 16

    h = _linear_bn_relu(z, lin_w, lin_b, bn0_g, bn0_b)
    x = h.reshape(B, C1, 16).transpose(1, 0, 2).reshape(C1, B * 16)

    p = _convt_bn_relu(x, w0_sub, b0, bn1_g, bn1_b, H=4, use_bn=True, act="relu")
    x = _interleave(p, B, 4)
    p = _convt_bn_relu(x, w1_sub, b1, bn2_g, bn2_b, H=8, use_bn=True, act="relu")
    x = _interleave(p, B, 8)
    p = _conv_last(x, w2_sub, b2, H=16)

    o = p.reshape(2, 2, B, 16, 16)
    o = jnp.transpose(o, (2, 3, 0, 4, 1))
    return o.reshape(B, 1, 32, 32)


# speedup vs baseline: 10.8137x; 1.5579x over previous
import functools

import jax
import jax.numpy as jnp
from jax.experimental import pallas as pl
from jax.experimental.pallas import tpu as pltpu

_BN_EPS = 1e-5
_TAPS = ((0, 0), (0, 1), (1, 0), (1, 1))
_HB = 4


def _linear_kernel(z_ref, w_ref, b_ref, g_ref, beta_ref, o_ref, *, batch):
    acc = jnp.dot(z_ref[...], w_ref[...], preferred_element_type=jnp.float32)
    acc = acc + b_ref[...]
    inv_n = 1.0 / batch
    mean = jnp.sum(acc, axis=0, keepdims=True) * inv_n
    var = jnp.sum((acc - mean) ** 2, axis=0, keepdims=True) * inv_n
    scale = g_ref[...] * jax.lax.rsqrt(var + _BN_EPS)
    shift = beta_ref[...] - mean * scale
    o_ref[...] = jnp.maximum(acc * scale + shift, 0.0)


def _linear_bn_relu(z, w, b, gamma, beta):
    B, K = z.shape
    N = w.shape[1]
    nb = 2 if N % 2 == 0 else 1
    Nb = N // nb
    return pl.pallas_call(
        functools.partial(_linear_kernel, batch=float(B)),
        out_shape=jax.ShapeDtypeStruct((B, N), jnp.float32),
        grid=(nb,),
        in_specs=[
            pl.BlockSpec((B, K), lambda i: (0, 0)),
            pl.BlockSpec((K, Nb), lambda i: (0, i)),
            pl.BlockSpec((1, Nb), lambda i: (0, i)),
            pl.BlockSpec((1, Nb), lambda i: (0, i)),
            pl.BlockSpec((1, Nb), lambda i: (0, i)),
        ],
        out_specs=pl.BlockSpec((B, Nb), lambda i: (0, i)),
        compiler_params=pltpu.CompilerParams(dimension_semantics=("parallel",)),
    )(z, w, b.reshape(1, N), gamma.reshape(1, N), beta.reshape(1, N))


def _shifted_chunks(rows, q, dyx, masks):
    dy, dx = dyx
    out = []
    for qy in range(q):
        sy, ry = qy + dy, 0
        if sy < 0:
            sy, ry = q - 1, -1
        elif sy >= q:
            sy, ry = 0, 1
        for qx in range(q):
            sx, rx = qx + dx, 0
            if sx < 0:
                sx, rx = q - 1, -1
            elif sx >= q:
                sx, rx = 0, 1
            c = rows[sy * q + sx]
            s = ry * _HB + rx
            if s:
                c = jnp.roll(c, -s, axis=1)
                c = jnp.where(masks[(ry, rx)], c, 0.0)
            out.append(c)
    return out


def _conv_level(planes, w_ref, cin, q, masks):
    xcat = jnp.concatenate(planes, axis=1).astype(jnp.bfloat16) \
        if len(planes) > 1 else planes[0].astype(jnp.bfloat16)
    m = planes[0].shape[1]
    accs = []
    for r in range(4):
        ry, rx = r // 2, r % 2
        wr = w_ref[r]
        acc = None
        for t, (dy, dx) in enumerate(_TAPS):
            wt = wr[:, t * cin:(t + 1) * cin].astype(jnp.bfloat16)
            y = jnp.dot(wt, xcat, preferred_element_type=jnp.float32)
            rows = [y[:, i * m:(i + 1) * m] for i in range(q * q)]
            ch = _shifted_chunks(rows, q, (ry + dy - 1, rx + dx - 1), masks)
            acc = ch if acc is None else [a + c for a, c in zip(acc, ch)]
        accs.append(acc)
    return accs


def _bn_relu_chunks(accs, b_ref, g_ref, beta_ref, n_total):
    flat = [c for acc in accs for c in acc]
    bias = b_ref[...]
    inv_n = 1.0 / n_total
    total = None
    for c in flat:
        s = (c + bias).sum(axis=1, keepdims=True)
        total = s if total is None else total + s
    mean = total * inv_n
    sq = None
    for c in flat:
        s = ((c + bias - mean) ** 2).sum(axis=1, keepdims=True)
        sq = s if sq is None else sq + s
    var = sq * inv_n
    scale = g_ref[...] * jax.lax.rsqrt(var + _BN_EPS)
    shift = beta_ref[...] + (bias - mean) * scale
    return [[jnp.maximum(c * scale + shift, 0.0) for c in acc] for acc in accs]


def _interleave_planes(accs, q):
    out = []
    for py in range(2 * q):
        qy, ry = py // 2, py % 2
        for px in range(2 * q):
            qx, rx = px // 2, px % 2
            out.append(accs[ry * 2 + rx][qy * q + qx])
    return out


def _mega_kernel(x0_ref, w0_ref, b0_ref, g1_ref, bt1_ref,
                 w1_ref, b1_ref, g2_ref, bt2_ref, w2_ref, o_ref, *,
                 c1, c2, c3):
    m = x0_ref.shape[1]
    lane = jax.lax.broadcasted_iota(jnp.int32, (1, m), 1)
    px = lane % _HB
    py = (lane // _HB) % _HB
    masks = {
        (0, 1): px + 1 < _HB, (0, -1): px >= 1,
        (1, 0): py + 1 < _HB, (-1, 0): py >= 1,
        (1, 1): (py + 1 < _HB) & (px + 1 < _HB),
        (1, -1): (py + 1 < _HB) & (px >= 1),
        (-1, 1): (py >= 1) & (px + 1 < _HB),
        (-1, -1): (py >= 1) & (px >= 1),
    }

    accs = _conv_level([x0_ref[...]], w0_ref, c1, 1, masks)
    accs = _bn_relu_chunks(accs, b0_ref, g1_ref, bt1_ref, float(4 * m))
    planes = _interleave_planes(accs, 1)

    accs = _conv_level(planes, w1_ref, c2, 2, masks)
    accs = _bn_relu_chunks(accs, b1_ref, g2_ref, bt2_ref, float(16 * m))
    planes = _interleave_planes(accs, 2)

    xcat = jnp.concatenate(planes, axis=1).astype(jnp.bfloat16)
    wall = w2_ref[:16, :].astype(jnp.bfloat16)
    ys = jnp.dot(wall, xcat, preferred_element_type=jnp.float32)
    bias = w2_ref[16, 0]
    q = 4
    final = []
    for r in range(4):
        ry, rx = r // 2, r % 2
        acc = None
        for t, (dy, dx) in enumerate(_TAPS):
            row = ys[4 * r + t:4 * r + t + 1, :]
            rows = [row[:, i * m:(i + 1) * m] for i in range(q * q)]
            ch = _shifted_chunks(rows, q, (ry + dy - 1, rx + dx - 1), masks)
            acc = ch if acc is None else [a + c for a, c in zip(acc, ch)]
        final.append(acc)
    out_rows = []
    for yb in range(8):
        pyy, ryy = yb // 2, yb % 2
        for xb in range(8):
            pxx, rxx = xb // 2, xb % 2
            out_rows.append(final[ryy * 2 + rxx][pyy * q + pxx])
    o_ref[...] = jnp.tanh(jnp.concatenate(out_rows, axis=0) + bias)


def _conv_stack(x0, w0_sub, b0, bn1_g, bn1_b, w1_sub, b1, bn2_g, bn2_b,
                w2_sub, b2):
    c1, m = x0.shape
    c2 = w0_sub.shape[1]
    c3 = w1_sub.shape[1]
    w2pack = jnp.concatenate(
        [w2_sub.reshape(16, c3), jnp.full((1, c3), 0.0) + b2], axis=0)
    return pl.pallas_call(
        functools.partial(_mega_kernel, c1=c1, c2=c2, c3=c3),
        out_shape=jax.ShapeDtypeStruct((64, m), jnp.float32),
        grid=(1,),
        in_specs=[
            pl.BlockSpec((c1, m), lambda i: (0, 0)),
            pl.BlockSpec((4, c2, 4 * c1), lambda i: (0, 0, 0)),
            pl.BlockSpec((c2, 1), lambda i: (0, 0)),
            pl.BlockSpec((c2, 1), lambda i: (0, 0)),
            pl.BlockSpec((c2, 1), lambda i: (0, 0)),
            pl.BlockSpec((4, c3, 4 * c2), lambda i: (0, 0, 0)),
            pl.BlockSpec((c3, 1), lambda i: (0, 0)),
            pl.BlockSpec((c3, 1), lambda i: (0, 0)),
            pl.BlockSpec((c3, 1), lambda i: (0, 0)),
            pl.BlockSpec((17, c3), lambda i: (0, 0)),
        ],
        out_specs=pl.BlockSpec((64, m), lambda i: (0, 0)),
        compiler_params=pltpu.CompilerParams(dimension_semantics=("arbitrary",)),
    )(x0, w0_sub, b0.reshape(c2, 1), bn1_g.reshape(c2, 1), bn1_b.reshape(c2, 1),
      w1_sub, b1.reshape(c3, 1), bn2_g.reshape(c3, 1), bn2_b.reshape(c3, 1),
      w2pack)


def kernel(lin_w, lin_b, bn0_g, bn0_b, w0, w0_sub, b0, bn1_g, bn1_b,
           w1, w1_sub, b1, bn2_g, bn2_b, w2, w2_sub, b2, z):
    B = z.shape[0]
    lin_out = lin_w.shape[1]
    C1 = lin_out // 16

    h = _linear_bn_relu(z, lin_w, lin_b, bn0_g, bn0_b)
    x0 = h.reshape(B, C1, 16).transpose(1, 0, 2).reshape(C1, B * 16)

    p = _conv_stack(x0, w0_sub, b0, bn1_g, bn1_b, w1_sub, b1, bn2_g, bn2_b,
                    w2_sub, b2)

    o = p.reshape(8, 8, B, 4, 4)
    o = jnp.transpose(o, (2, 3, 0, 4, 1))
    return o.reshape(B, 1, 32, 32)


# speedup vs baseline: 10.8873x; 1.0068x over previous
import functools

import jax
import jax.numpy as jnp
from jax.experimental import pallas as pl
from jax.experimental.pallas import tpu as pltpu

_BN_EPS = 1e-5
_TAPS = ((0, 0), (0, 1), (1, 0), (1, 1))
_HB = 4


def _linear_kernel(z_ref, w_ref, b_ref, g_ref, beta_ref, o_ref, *, batch):
    acc = jnp.dot(z_ref[...], w_ref[...], preferred_element_type=jnp.float32)
    acc = acc + b_ref[...]
    inv_n = 1.0 / batch
    mean = jnp.sum(acc, axis=0, keepdims=True) * inv_n
    var = jnp.sum((acc - mean) ** 2, axis=0, keepdims=True) * inv_n
    scale = g_ref[...] * jax.lax.rsqrt(var + _BN_EPS)
    shift = beta_ref[...] - mean * scale
    o_ref[...] = jnp.maximum(acc * scale + shift, 0.0)


def _linear_bn_relu(z, w, b, gamma, beta):
    B, K = z.shape
    N = w.shape[1]
    nb = 2 if N % 2 == 0 else 1
    Nb = N // nb
    return pl.pallas_call(
        functools.partial(_linear_kernel, batch=float(B)),
        out_shape=jax.ShapeDtypeStruct((B, N), jnp.float32),
        grid=(nb,),
        in_specs=[
            pl.BlockSpec((B, K), lambda i: (0, 0)),
            pl.BlockSpec((K, Nb), lambda i: (0, i)),
            pl.BlockSpec((1, Nb), lambda i: (0, i)),
            pl.BlockSpec((1, Nb), lambda i: (0, i)),
            pl.BlockSpec((1, Nb), lambda i: (0, i)),
        ],
        out_specs=pl.BlockSpec((B, Nb), lambda i: (0, i)),
        compiler_params=pltpu.CompilerParams(dimension_semantics=("parallel",)),
    )(z, w, b.reshape(1, N), gamma.reshape(1, N), beta.reshape(1, N))


def _shifted_chunks(rows, q, dyx, masks):
    dy, dx = dyx
    out = []
    for qy in range(q):
        sy, ry = qy + dy, 0
        if sy < 0:
            sy, ry = q - 1, -1
        elif sy >= q:
            sy, ry = 0, 1
        for qx in range(q):
            sx, rx = qx + dx, 0
            if sx < 0:
                sx, rx = q - 1, -1
            elif sx >= q:
                sx, rx = 0, 1
            c = rows[sy * q + sx]
            s = ry * _HB + rx
            if s:
                c = jnp.roll(c, -s, axis=1)
                c = jnp.where(masks[(ry, rx)], c, 0.0)
            out.append(c)
    return out


def _conv_level(planes, w_ref, cin, q, masks, wait_fn=None):
    xcat = jnp.concatenate(planes, axis=1).astype(jnp.bfloat16) \
        if len(planes) > 1 else planes[0].astype(jnp.bfloat16)
    m = planes[0].shape[1]
    accs = []
    for r in range(4):
        ry, rx = r // 2, r % 2
        if wait_fn is not None:
            wait_fn(r)
        wr = w_ref[r]
        acc = None
        for t, (dy, dx) in enumerate(_TAPS):
            wt = wr[:, t * cin:(t + 1) * cin].astype(jnp.bfloat16)
            y = jnp.dot(wt, xcat, preferred_element_type=jnp.float32)
            rows = [y[:, i * m:(i + 1) * m] for i in range(q * q)]
            ch = _shifted_chunks(rows, q, (ry + dy - 1, rx + dx - 1), masks)
            acc = ch if acc is None else [a + c for a, c in zip(acc, ch)]
        accs.append(acc)
    return accs


def _bn_relu_chunks(accs, b_ref, g_ref, beta_ref, n_total):
    flat = [c for acc in accs for c in acc]
    bias = b_ref[...]
    inv_n = 1.0 / n_total
    total = None
    for c in flat:
        s = (c + bias).sum(axis=1, keepdims=True)
        total = s if total is None else total + s
    mean = total * inv_n
    sq = None
    for c in flat:
        s = ((c + bias - mean) ** 2).sum(axis=1, keepdims=True)
        sq = s if sq is None else sq + s
    var = sq * inv_n
    scale = g_ref[...] * jax.lax.rsqrt(var + _BN_EPS)
    shift = beta_ref[...] + (bias - mean) * scale
    return [[jnp.maximum(c * scale + shift, 0.0) for c in acc] for acc in accs]


def _interleave_planes(accs, q):
    out = []
    for py in range(2 * q):
        qy, ry = py // 2, py % 2
        for px in range(2 * q):
            qx, rx = px // 2, px % 2
            out.append(accs[ry * 2 + rx][qy * q + qx])
    return out


def _mega_kernel(x0_ref, w0_hbm, b0_ref, g1_ref, bt1_ref,
                 w1_hbm, b1_ref, g2_ref, bt2_ref, w2_ref, o_ref,
                 w0_ref, w1_ref, sems, *, c1, c2, c3):
    for r in range(4):
        pltpu.make_async_copy(w0_hbm.at[r], w0_ref.at[r], sems.at[r]).start()
    pltpu.make_async_copy(w1_hbm, w1_ref, sems.at[4]).start()

    def wait_w0(r):
        pltpu.make_async_copy(w0_hbm.at[r], w0_ref.at[r], sems.at[r]).wait()

    m = x0_ref.shape[1]
    lane = jax.lax.broadcasted_iota(jnp.int32, (1, m), 1)
    px = lane % _HB
    py = (lane // _HB) % _HB
    masks = {
        (0, 1): px + 1 < _HB, (0, -1): px >= 1,
        (1, 0): py + 1 < _HB, (-1, 0): py >= 1,
        (1, 1): (py + 1 < _HB) & (px + 1 < _HB),
        (1, -1): (py + 1 < _HB) & (px >= 1),
        (-1, 1): (py >= 1) & (px + 1 < _HB),
        (-1, -1): (py >= 1) & (px >= 1),
    }

    accs = _conv_level([x0_ref[...]], w0_ref, c1, 1, masks, wait_fn=wait_w0)
    accs = _bn_relu_chunks(accs, b0_ref, g1_ref, bt1_ref, float(4 * m))
    planes = _interleave_planes(accs, 1)

    pltpu.make_async_copy(w1_hbm, w1_ref, sems.at[4]).wait()
    accs = _conv_level(planes, w1_ref, c2, 2, masks)
    accs = _bn_relu_chunks(accs, b1_ref, g2_ref, bt2_ref, float(16 * m))
    planes = _interleave_planes(accs, 2)

    xcat = jnp.concatenate(planes, axis=1).astype(jnp.bfloat16)
    wall = w2_ref[:16, :].astype(jnp.bfloat16)
    ys = jnp.dot(wall, xcat, preferred_element_type=jnp.float32)
    bias = w2_ref[16, 0]
    q = 4
    final = []
    for r in range(4):
        ry, rx = r // 2, r % 2
        acc = None
        for t, (dy, dx) in enumerate(_TAPS):
            row = ys[4 * r + t:4 * r + t + 1, :]
            rows = [row[:, i * m:(i + 1) * m] for i in range(q * q)]
            ch = _shifted_chunks(rows, q, (ry + dy - 1, rx + dx - 1), masks)
            acc = ch if acc is None else [a + c for a, c in zip(acc, ch)]
        final.append(acc)
    out_rows = []
    for yb in range(8):
        pyy, ryy = yb // 2, yb % 2
        for xb in range(8):
            pxx, rxx = xb // 2, xb % 2
            out_rows.append(final[ryy * 2 + rxx][pyy * q + pxx])
    o_ref[...] = jnp.tanh(jnp.concatenate(out_rows, axis=0) + bias)


def _conv_stack(x0, w0_sub, b0, bn1_g, bn1_b, w1_sub, b1, bn2_g, bn2_b,
                w2_sub, b2):
    c1, m = x0.shape
    c2 = w0_sub.shape[1]
    c3 = w1_sub.shape[1]
    w2pack = jnp.concatenate(
        [w2_sub.reshape(16, c3), jnp.full((1, c3), 0.0) + b2], axis=0)
    return pl.pallas_call(
        functools.partial(_mega_kernel, c1=c1, c2=c2, c3=c3),
        out_shape=jax.ShapeDtypeStruct((64, m), jnp.float32),
        grid=(1,),
        in_specs=[
            pl.BlockSpec((c1, m), lambda i: (0, 0)),
            pl.BlockSpec(memory_space=pltpu.MemorySpace.HBM),
            pl.BlockSpec((c2, 1), lambda i: (0, 0)),
            pl.BlockSpec((c2, 1), lambda i: (0, 0)),
            pl.BlockSpec((c2, 1), lambda i: (0, 0)),
            pl.BlockSpec(memory_space=pltpu.MemorySpace.HBM),
            pl.BlockSpec((c3, 1), lambda i: (0, 0)),
            pl.BlockSpec((c3, 1), lambda i: (0, 0)),
            pl.BlockSpec((c3, 1), lambda i: (0, 0)),
            pl.BlockSpec((17, c3), lambda i: (0, 0)),
        ],
        out_specs=pl.BlockSpec((64, m), lambda i: (0, 0)),
        scratch_shapes=[
            pltpu.VMEM((4, c2, 4 * c1), jnp.float32),
            pltpu.VMEM((4, c3, 4 * c2), jnp.float32),
            pltpu.SemaphoreType.DMA((5,)),
        ],
        compiler_params=pltpu.CompilerParams(dimension_semantics=("arbitrary",)),
    )(x0, w0_sub, b0.reshape(c2, 1), bn1_g.reshape(c2, 1), bn1_b.reshape(c2, 1),
      w1_sub, b1.reshape(c3, 1), bn2_g.reshape(c3, 1), bn2_b.reshape(c3, 1),
      w2pack)


def kernel(lin_w, lin_b, bn0_g, bn0_b, w0, w0_sub, b0, bn1_g, bn1_b,
           w1, w1_sub, b1, bn2_g, bn2_b, w2, w2_sub, b2, z):
    B = z.shape[0]
    lin_out = lin_w.shape[1]
    C1 = lin_out // 16

    h = _linear_bn_relu(z, lin_w, lin_b, bn0_g, bn0_b)
    x0 = h.reshape(B, C1, 16).transpose(1, 0, 2).reshape(C1, B * 16)

    p = _conv_stack(x0, w0_sub, b0, bn1_g, bn1_b, w1_sub, b1, bn2_g, bn2_b,
                    w2_sub, b2)

    o = p.reshape(8, 8, B, 4, 4)
    o = jnp.transpose(o, (2, 3, 0, 4, 1))
    return o.reshape(B, 1, 32, 32)
